# Initial kernel scaffold; baseline (speedup 1.0000x reference)
import jax
import jax.numpy as jnp
from jax.experimental import pallas as pl


def kernel(messages, emb, W_track, b_track, W_trans, b_trans, W_comp, b_comp, ln_g, ln_b):
    raise NotImplementedError("write your pallas kernel here")


# speedup vs baseline: 5.6180x; 5.6180x over previous
import functools

import jax
import jax.numpy as jnp
import numpy as np
from jax import lax
from jax.experimental import pallas as pl
from jax.experimental.pallas import tpu as pltpu
from jax.experimental.pallas import tpu_sc as plsc

D_VEC = 128
D_TRACK = 64
L = 50
T_TOT = 2 * L - 1
FMIN = float(np.finfo(np.float32).min)
BB = 128


def _sc_gather(emb, ids):
    n = ids.shape[0]
    d = emb.shape[1]
    info = plsc.get_sparse_core_info()
    nw = info.num_cores * info.num_subcores
    b_per_w = n // nw
    cb = 80
    n_chunks = b_per_w // cb
    mesh = plsc.VectorSubcoreMesh(core_axis_name="c", subcore_axis_name="s")

    @functools.partial(
        pl.kernel,
        out_type=jax.ShapeDtypeStruct((n, d), jnp.float32),
        mesh=mesh,
        scratch_types=[
            pltpu.VMEM((cb,), jnp.int32),
            pltpu.VMEM((cb, d), jnp.float32),
            pltpu.SemaphoreType.DMA,
        ],
    )
    def gather_kernel(table_hbm, idx_hbm, out_hbm, idx_v, rows_v, sem):
        wid = lax.axis_index("s") * info.num_cores + lax.axis_index("c")
        base = wid * b_per_w
        for j in range(n_chunks):
            off = base + j * cb
            pltpu.sync_copy(idx_hbm.at[pl.ds(off, cb)], idx_v)
            pltpu.async_copy(table_hbm.at[idx_v], rows_v, sem).wait()
            pltpu.sync_copy(rows_v, out_hbm.at[pl.ds(off, cb)])

    return gather_kernel(emb, ids)


def _mm(x, w):
    return lax.dot_general(
        x, w, (((1,), (0,)), ((), ())),
        preferred_element_type=jnp.float32,
    )


def _mm_k448(x, w):
    return _mm(x[:, :256], w[:256]) + _mm(x[:, 256:], w[256:])


def _rowsum128(v):
    rr = v
    acc = v
    for _ in range(15):
        rr = pltpu.roll(rr, 120, 1)
        acc = acc + rr
    acc = acc + pltpu.roll(acc, 124, 1)
    acc = acc + pltpu.roll(acc, 126, 1)
    acc = acc + pltpu.roll(acc, 127, 1)
    return acc[:, 0:1]


def _rowsum256(x):
    return _rowsum128(x[:, :128] + x[:, 128:])


def _loop_kernel(reps_ref, msg_ref, wt_ref, bt_ref, wtr_ref,
                 btr_ref, wc_ref, bc_ref, g_ref, b_ref,
                 hidt_ref, hid_ref, tr_ref, lp_ref, en_ref):
    g = g_ref[...]
    b = b_ref[...]

    def ln(x):
        mu = _rowsum256(x) * (1.0 / 256.0)
        dx = x - mu
        var = _rowsum256(dx * dx) * (1.0 / 256.0)
        return dx / jnp.sqrt(var + 1e-5) * g + b

    msg = msg_ref[...]
    mlen = jnp.sum((msg != 0).astype(jnp.int32), axis=1, keepdims=True)

    e0 = ln(reps_ref[0])
    e1 = ln(reps_ref[1])
    hidt_ref[0] = e0
    hidt_ref[1] = e1

    ts = lax.broadcasted_iota(jnp.int32, (1, 128), 1)
    tracc = jnp.where((ts <= 1) & (ts >= 2 * mlen - 1), 2,
                      jnp.zeros((BB, 128), jnp.int32))
    lpacc = jnp.zeros((BB, 128), jnp.float32)
    enacc = jnp.zeros((BB, 128), jnp.float32)

    tlast = 2 * mlen - 2
    hid = jnp.where(tlast == 0, e0[:, :D_VEC],
                    jnp.where(tlast == 1, e1[:, :D_VEC],
                              jnp.zeros((BB, D_VEC), jnp.float32)))

    wt = wt_ref[...]
    wt1 = wt[0:128]
    wt2 = wt[128:256]
    wt3 = wt[256:384]
    wt4 = wt[384:448]
    bt = bt_ref[...]
    wtr = wtr_ref[...]
    btr = btr_ref[...]
    wc = wc_ref[...]
    bc = bc_ref[...]

    th0 = jnp.zeros((BB, D_TRACK), jnp.float32)
    tc0 = jnp.zeros((BB, D_TRACK), jnp.float32)
    qi0 = jnp.full((BB, 1), 1, jnp.int32)
    bp0 = jnp.full((BB, 1), 2, jnp.int32)
    carry0 = (th0, tc0, e1[:, :D_VEC], e0[:, :D_VEC], reps_ref[2],
              qi0, bp0, mlen - 2, hid, tracc, lpacc, enacc)

    def step(t, c):
        (th, tcell, tos1, tos2, cur_buf, qi, bp, unexec,
         hid, tracc, lpacc, enacc) = c
        top_buf = cur_buf[:, :D_VEC]
        x = jnp.concatenate([top_buf, tos1, tos2, th], axis=1)
        gates = _mm_k448(x, wt) + bt
        ig_ = jax.nn.sigmoid(gates[:, 0:64])
        fg_ = jax.nn.sigmoid(gates[:, 64:128])
        gg_ = jnp.tanh(gates[:, 128:192])
        og_ = jax.nn.sigmoid(gates[:, 192:256])
        tcell = fg_ * tcell + ig_ * gg_
        th = og_ * jnp.tanh(tcell)

        logits = _mm(th, wtr) + btr
        l0 = logits[:, 0:1]
        l1 = logits[:, 1:2]
        l0m = l0 + (unexec == 0).astype(jnp.float32) * FMIN
        l1m = l1 + (qi <= 0).astype(jnp.float32) * FMIN
        red = l1m > l0m
        m = jnp.maximum(l0m, l1m)
        s0 = l0m - m
        s1 = l1m - m
        ls = jnp.log(jnp.exp(s0) + jnp.exp(s1))
        lsm0 = s0 - ls
        lsm1 = s1 - ls
        lp = jnp.where(red, lsm1, lsm0)
        p0 = jnp.exp(lsm0)
        p1 = jnp.exp(lsm1)
        ent = -(jnp.where(p0 > 0, p0 * lsm0, 0.0)
                + jnp.where(p1 > 0, p1 * lsm1, 0.0))

        redi = red.astype(jnp.int32)
        qi = jnp.clip(qi + 1 - 2 * redi, -1, L - 1)
        bp = jnp.minimum(bp + 1 - redi, L - 1)
        unexec = unexec - (1 - redi)

        gc = _mm(th, wc) + bc
        cig = jax.nn.sigmoid(gc[:, 0:128])
        cog = jax.nn.sigmoid(gc[:, 128:256])
        cgg = jnp.tanh(gc[:, 256:384])
        cc = cig * cgg
        hh = cog * cc

        nb = cur_buf
        for pos in range(2, L):
            nb = jnp.where(bp == pos, reps_ref[pos], nb)

        new_entry = jnp.where(red, jnp.concatenate([hh, cc], axis=1), nb)
        entln = ln(new_entry)
        hidt_ref[pl.ds(t, 1)] = entln[None]

        skipm = t >= (2 * mlen - 1)
        sel = ts == t
        tracc = jnp.where(sel, jnp.where(skipm, 2, redi), tracc)
        lpacc = jnp.where(sel, jnp.where(skipm, 0.0, lp), lpacc)
        enacc = jnp.where(sel, jnp.where(skipm, 0.0, ent), enacc)
        hid = jnp.where(t == tlast, entln[:, :D_VEC], hid)

        return (th, tcell, entln[:, :D_VEC], tos1, nb, qi, bp, unexec,
                hid, tracc, lpacc, enacc)

    carry = lax.fori_loop(2, T_TOT, step, carry0)
    (_, _, _, _, _, _, _, _, hid, tracc, lpacc, enacc) = carry
    hid_ref[...] = hid
    tr_ref[...] = tracc[:, :T_TOT]
    lp_ref[...] = lpacc[:, :T_TOT]
    en_ref[...] = enacc[:, :T_TOT]


def _run_loop(reps_t, msgs, wt, bt, wtr, btr, wc, bc, g2, b2):
    bat = msgs.shape[0]
    grid = bat // BB
    return pl.pallas_call(
        _loop_kernel,
        grid=(grid,),
        in_specs=[
            pl.BlockSpec((L, BB, 2 * D_VEC), lambda i: (0, i, 0)),
            pl.BlockSpec((BB, L), lambda i: (i, 0)),
            pl.BlockSpec((3 * D_VEC + D_TRACK, 4 * D_TRACK), lambda i: (0, 0)),
            pl.BlockSpec((1, 4 * D_TRACK), lambda i: (0, 0)),
            pl.BlockSpec((D_TRACK, 2), lambda i: (0, 0)),
            pl.BlockSpec((1, 2), lambda i: (0, 0)),
            pl.BlockSpec((D_TRACK, 3 * D_VEC), lambda i: (0, 0)),
            pl.BlockSpec((1, 3 * D_VEC), lambda i: (0, 0)),
            pl.BlockSpec((1, 2 * D_VEC), lambda i: (0, 0)),
            pl.BlockSpec((1, 2 * D_VEC), lambda i: (0, 0)),
        ],
        out_specs=[
            pl.BlockSpec((T_TOT, BB, 2 * D_VEC), lambda i: (0, i, 0)),
            pl.BlockSpec((BB, D_VEC), lambda i: (i, 0)),
            pl.BlockSpec((BB, T_TOT), lambda i: (i, 0)),
            pl.BlockSpec((BB, T_TOT), lambda i: (i, 0)),
            pl.BlockSpec((BB, T_TOT), lambda i: (i, 0)),
        ],
        out_shape=[
            jax.ShapeDtypeStruct((T_TOT, bat, 2 * D_VEC), jnp.float32),
            jax.ShapeDtypeStruct((bat, D_VEC), jnp.float32),
            jax.ShapeDtypeStruct((bat, T_TOT), jnp.int32),
            jax.ShapeDtypeStruct((bat, T_TOT), jnp.float32),
            jax.ShapeDtypeStruct((bat, T_TOT), jnp.float32),
        ],
    )(reps_t, msgs, wt, bt, wtr, btr, wc, bc, g2, b2)


def kernel(messages, emb, W_track, b_track, W_trans, b_trans,
           W_comp, b_comp, ln_g, ln_b):
    bat = messages.shape[0]
    msgs = messages.astype(jnp.int32)

    ids_lmajor = jnp.transpose(msgs).reshape(-1)
    reps_t = _sc_gather(emb, ids_lmajor).reshape(L, bat, 2 * D_VEC)

    wt = W_track.T
    bt = b_track.reshape(1, -1)
    wtr = W_trans.T
    btr = b_trans.reshape(1, 2)
    wc_full = W_comp[:, 2 * D_VEC:]
    wc = jnp.concatenate(
        [wc_full[0:128], wc_full[384:512], wc_full[512:640]], axis=0).T
    bc = jnp.concatenate(
        [b_comp[0:128], b_comp[384:512], b_comp[512:640]]).reshape(1, -1)
    g2 = ln_g.reshape(1, -1)
    b2 = ln_b.reshape(1, -1)

    hidt, hidden, tr, lp, en = _run_loop(
        reps_t, msgs, wt, bt, wtr, btr, wc, bc, g2, b2)
    hiddens = jnp.transpose(hidt, (1, 0, 2))
    return hidden, hiddens, tr, lp, en
